# baseline (speedup 1.0000x reference)
import jax
import jax.numpy as jnp
from jax import lax
from jax.experimental import pallas as pl
from jax.experimental.pallas import tpu as pltpu
from jax.experimental.pallas import tpu_sc as plsc

_NUM_CORES = 2
_NUM_SUBCORES = 16
_NUM_WORKERS = _NUM_CORES * _NUM_SUBCORES
_LANES = 16
_NBUF = 4
_LOOKAHEAD = 2
_ROW = 128


def _build_sc_embed(batch, maxlen, embed):
    assert batch % _NUM_WORKERS == 0
    assert embed % _LANES == 0 and embed <= _ROW
    chunk = 40
    assert maxlen % chunk == 0 and chunk % 8 == 0
    cps = maxlen // chunk
    seq_per_worker = batch // _NUM_WORKERS
    chunks_per_worker = seq_per_worker * cps
    mesh = plsc.VectorSubcoreMesh(core_axis_name="c", subcore_axis_name="s")

    @jax.jit
    def run(inputs, token_table_padded, pos_table):
        @pl.kernel(
            out_type=jax.ShapeDtypeStruct((batch, maxlen, embed), jnp.float32),
            mesh=mesh,
            compiler_params=pltpu.CompilerParams(
                use_tc_tiling_on_sc=False, skip_device_barrier=True),
            scratch_types=[
                pltpu.VMEM((seq_per_worker * maxlen,), jnp.int32),
                pltpu.VMEM((maxlen, embed), jnp.float32),
                pltpu.VMEM((_NBUF, chunk, _ROW), jnp.float32),
                pltpu.SemaphoreType.DMA((_NBUF,)),
                pltpu.SemaphoreType.DMA((_NBUF,)),
            ],
        )
        def body(idx_hbm, table_hbm, pos_hbm, out_hbm, idx_v, pos_v, rows_v,
                 gsem, osem):
            wid = lax.axis_index("s") * _NUM_CORES + lax.axis_index("c")
            seq0 = wid * seq_per_worker

            pltpu.sync_copy(pos_hbm, pos_v)
            pltpu.sync_copy(
                idx_hbm.at[pl.ds(seq0 * maxlen, seq_per_worker * maxlen)],
                idx_v)

            def gather_start(h, slot):
                return pltpu.async_copy(
                    table_hbm.at[idx_v.at[pl.ds(h * chunk, chunk)]],
                    rows_v.at[slot], gsem.at[slot])

            gather_h = [None] * _NBUF
            out_h = [None] * _NBUF
            for h in range(min(_LOOKAHEAD, chunks_per_worker)):
                gather_h[h % _NBUF] = gather_start(h, h % _NBUF)

            for h in range(chunks_per_worker):
                slot = h % _NBUF
                nh = h + _LOOKAHEAD
                if nh < chunks_per_worker:
                    nslot = nh % _NBUF
                    if out_h[nslot] is not None:
                        out_h[nslot].wait()
                        out_h[nslot] = None
                    gather_h[nslot] = gather_start(nh, nslot)

                gather_h[slot].wait()
                gather_h[slot] = None
                buf = rows_v.at[slot]
                pos_base = (h % cps) * chunk

                @pl.loop(0, chunk)
                def _(r):
                    for c in range(0, embed, _LANES):
                        src = (pl.ds(pos_base + r, 1), pl.ds(c, _LANES))
                        dst = (pl.ds(r, 1), pl.ds(c, _LANES))
                        buf.at[dst][...] = buf.at[dst][...] + pos_v.at[src][...]

                out_h[slot] = pltpu.async_copy(
                    buf.at[:, pl.ds(0, embed)],
                    out_hbm.at[seq0 + h // cps, pl.ds(pos_base, chunk)],
                    osem.at[slot])

            for hnd in out_h:
                if hnd is not None:
                    hnd.wait()

        return body(inputs, token_table_padded, pos_table)

    return run


def kernel(inputs, token_table, pos_table):
    batch, maxlen = inputs.shape
    _, embed = token_table.shape
    ids = inputs.astype(jnp.int32).reshape(-1)
    tpad = jnp.pad(token_table, ((0, 0), (0, _ROW - embed)))
    run = _build_sc_embed(batch, maxlen, embed)
    return run(ids, tpad, pos_table)


# speedup vs baseline: 1.0738x; 1.0738x over previous
import jax
import jax.numpy as jnp
from jax import lax
from jax.experimental import pallas as pl
from jax.experimental.pallas import tpu as pltpu
from jax.experimental.pallas import tpu_sc as plsc

_NUM_CORES = 2
_NUM_SUBCORES = 16
_NUM_WORKERS = _NUM_CORES * _NUM_SUBCORES
_LANES = 16
_NBUF = 4
_LOOKAHEAD = 2


def _build_sc_embed(batch, maxlen, embed):
    assert batch % _NUM_WORKERS == 0
    assert embed % _LANES == 0
    seq_per_worker = batch // _NUM_WORKERS
    mesh = plsc.VectorSubcoreMesh(core_axis_name="c", subcore_axis_name="s")

    @jax.jit
    def run(inputs, token_table, pos_table):
        @pl.kernel(
            out_type=jax.ShapeDtypeStruct((batch, maxlen, embed), jnp.float32),
            mesh=mesh,
            compiler_params=pltpu.CompilerParams(
                use_tc_tiling_on_sc=False, skip_device_barrier=True),
            scratch_types=[
                pltpu.VMEM((seq_per_worker * maxlen,), jnp.int32),
                pltpu.VMEM((maxlen, embed), jnp.float32),
                pltpu.VMEM((_NBUF, maxlen, embed), jnp.float32),
                pltpu.SemaphoreType.DMA((_NBUF,)),
                pltpu.SemaphoreType.DMA((_NBUF,)),
            ],
        )
        def body(idx_hbm, table_hbm, pos_hbm, out_hbm, idx_v, pos_v, rows_v,
                 gsem, osem):
            wid = lax.axis_index("s") * _NUM_CORES + lax.axis_index("c")
            row0 = wid * seq_per_worker

            pltpu.sync_copy(pos_hbm, pos_v)
            pltpu.sync_copy(
                idx_hbm.at[pl.ds(row0 * maxlen, seq_per_worker * maxlen)],
                idx_v)

            def gather_start(j, slot):
                return pltpu.async_copy(
                    table_hbm.at[idx_v.at[pl.ds(j * maxlen, maxlen)]],
                    rows_v.at[slot], gsem.at[slot])

            gather_h = [None] * _NBUF
            out_h = [None] * _NBUF
            for j in range(min(_LOOKAHEAD, seq_per_worker)):
                gather_h[j % _NBUF] = gather_start(j, j % _NBUF)

            for j in range(seq_per_worker):
                slot = j % _NBUF
                nj = j + _LOOKAHEAD
                if nj < seq_per_worker:
                    nslot = nj % _NBUF
                    if out_h[nslot] is not None:
                        out_h[nslot].wait()
                        out_h[nslot] = None
                    gather_h[nslot] = gather_start(nj, nslot)

                gather_h[slot].wait()
                gather_h[slot] = None
                buf = rows_v.at[slot]

                @pl.loop(0, maxlen)
                def _(r):
                    for c in range(0, embed, _LANES):
                        slc = (pl.ds(r, 1), pl.ds(c, _LANES))
                        buf.at[slc][...] = buf.at[slc][...] + pos_v.at[slc][...]

                out_h[slot] = pltpu.async_copy(
                    buf, out_hbm.at[row0 + j], osem.at[slot])

            for hnd in out_h:
                if hnd is not None:
                    hnd.wait()

        return body(inputs, token_table, pos_table)

    return run


def kernel(inputs, token_table, pos_table):
    batch, maxlen = inputs.shape
    _, embed = token_table.shape
    ids = inputs.astype(jnp.int32).reshape(-1)
    run = _build_sc_embed(batch, maxlen, embed)
    return run(ids, token_table, pos_table)


# speedup vs baseline: 1.5979x; 1.4881x over previous
import jax
import jax.numpy as jnp
from jax import lax
from jax.experimental import pallas as pl
from jax.experimental.layout import Format, Layout
from jax.experimental.pallas import tpu as pltpu
from jax.experimental.pallas import tpu_sc as plsc

_NUM_CORES = 2
_NUM_SUBCORES = 16
_NUM_WORKERS = _NUM_CORES * _NUM_SUBCORES
_LANES = 16
_NBUF = 4
_LOOKAHEAD = 2


def _build_sc_embed(batch, maxlen, embed):
    assert batch % _NUM_WORKERS == 0
    assert embed % _LANES == 0
    seq_per_worker = batch // _NUM_WORKERS
    mesh = plsc.VectorSubcoreMesh(core_axis_name="c", subcore_axis_name="s")

    @jax.jit
    def run(inputs, token_table, pos_table):
        from jax.experimental.layout import with_layout_constraint
        token_table = with_layout_constraint(
            token_table, Layout(major_to_minor=(0, 1), tiling=((8,),)))
        @pl.kernel(
            out_type=jax.ShapeDtypeStruct((batch, maxlen, embed), jnp.float32),
            mesh=mesh,
            compiler_params=pltpu.CompilerParams(
                use_tc_tiling_on_sc=False, skip_device_barrier=True),
            scratch_types=[
                pltpu.VMEM((seq_per_worker * maxlen,), jnp.int32),
                pltpu.VMEM((maxlen, embed), jnp.float32),
                pltpu.VMEM((_NBUF, maxlen, embed), jnp.float32),
                pltpu.SemaphoreType.DMA((_NBUF,)),
                pltpu.SemaphoreType.DMA((_NBUF,)),
            ],
        )
        def body(idx_hbm, table_hbm, pos_hbm, out_hbm, idx_v, pos_v, rows_v,
                 gsem, osem):
            wid = lax.axis_index("s") * _NUM_CORES + lax.axis_index("c")
            row0 = wid * seq_per_worker

            pltpu.sync_copy(pos_hbm, pos_v)
            pltpu.sync_copy(
                idx_hbm.at[pl.ds(row0 * maxlen, seq_per_worker * maxlen)],
                idx_v)

            def gather_start(j, slot):
                return pltpu.async_copy(
                    table_hbm.at[idx_v.at[pl.ds(j * maxlen, maxlen)]],
                    rows_v.at[slot], gsem.at[slot])

            gather_h = [None] * _NBUF
            out_h = [None] * _NBUF
            for j in range(min(_LOOKAHEAD, seq_per_worker)):
                gather_h[j % _NBUF] = gather_start(j, j % _NBUF)

            for j in range(seq_per_worker):
                slot = j % _NBUF
                nj = j + _LOOKAHEAD
                if nj < seq_per_worker:
                    nslot = nj % _NBUF
                    if out_h[nslot] is not None:
                        out_h[nslot].wait()
                        out_h[nslot] = None
                    gather_h[nslot] = gather_start(nj, nslot)

                gather_h[slot].wait()
                gather_h[slot] = None
                buf = rows_v.at[slot]

                @pl.loop(0, maxlen)
                def _(r):
                    for c in range(0, embed, _LANES):
                        slc = (pl.ds(r, 1), pl.ds(c, _LANES))
                        buf.at[slc][...] = buf.at[slc][...] + pos_v.at[slc][...]

                out_h[slot] = pltpu.async_copy(
                    buf, out_hbm.at[row0 + j], osem.at[slot])

            for hnd in out_h:
                if hnd is not None:
                    hnd.wait()

        return body(inputs, token_table, pos_table)

    return run


def kernel(inputs, token_table, pos_table):
    batch, maxlen = inputs.shape
    _, embed = token_table.shape
    ids = inputs.astype(jnp.int32).reshape(-1)
    run = _build_sc_embed(batch, maxlen, embed)
    return run(ids, token_table, pos_table)
